# baseline (speedup 1.0000x reference)
import functools

import jax
import jax.numpy as jnp
from jax import lax
from jax.experimental import pallas as pl
from jax.experimental.pallas import tpu as pltpu
from jax.experimental.pallas import tpu_sc as plsc

B, NP, NQ, D, K = 2, 4096, 4096, 256, 16
TQ = 256
SLICES_PER_B = 2
QS = NQ // SLICES_PER_B
NC, NS = 2, 16
NW = NC * NS
QW = QS // NW
CQ = 16
_PREC = lax.Precision.DEFAULT


def _normalize_body(x_ref, y_ref):
    x = x_ref[...]
    n = jnp.sqrt(jnp.sum(x * x, axis=1, keepdims=True))
    y_ref[...] = x / (n + 1e-8)


def _normalize(x):
    rows = x.shape[0]
    return pl.pallas_call(
        _normalize_body,
        grid=(rows // 1024,),
        in_specs=[pl.BlockSpec((1024, D), lambda i: (i, 0))],
        out_specs=pl.BlockSpec((1024, D), lambda i: (i, 0)),
        out_shape=jax.ShapeDtypeStruct((rows, D), jnp.float32),
    )(x)


_NB = 2
_GR = NP // _NB
_BIG = 1 << 30


def _topk_body(f1n_ref, f2n_ref, idx_ref):
    f1n = f1n_ref[...]
    f2n = f2n_ref[...]
    s = lax.dot_general(f1n, f2n, (((1,), (1,)), ((), ())),
                        preferred_element_type=jnp.float32,
                        precision=_PREC)
    riota = lax.broadcasted_iota(jnp.int32, (_GR, TQ), 0)
    vals = [s[j * _GR:(j + 1) * _GR, :] for j in range(_NB)]
    idxs = [riota + (j * _GR) for j in range(_NB)]

    def ce(a, b):
        (av, ai), (bv, bi) = a, b
        sw = bv > av
        return ((jnp.maximum(av, bv), jnp.where(sw, bi, ai)),
                (jnp.minimum(av, bv), jnp.where(sw, ai, bi)))

    r = list(zip(vals, idxs))
    r[0], r[1] = ce(r[0], r[1])
    (v1, i1), (v2, i2) = r

    for t in range(K):
        m = jnp.max(v1, axis=0)
        cand = jnp.where(v1 == m[None, :], i1, _BIG)
        sel = jnp.min(cand, axis=0)
        idx_ref[t, :] = sel
        hit = riota == (sel & (_GR - 1))[None, :]
        v1 = jnp.where(hit, v2, v1)
        i1 = jnp.where(hit, i2, i1)
        v2 = jnp.where(hit, -jnp.inf, v2)


def _topk_slice(f1nb, f2ns):
    return pl.pallas_call(
        _topk_body,
        grid=(QS // TQ,),
        in_specs=[
            pl.BlockSpec((NP, D), lambda i: (0, 0)),
            pl.BlockSpec((TQ, D), lambda i: (i, 0)),
        ],
        out_specs=pl.BlockSpec((K, TQ), lambda i: (0, i)),
        out_shape=jax.ShapeDtypeStruct((K, QS), jnp.int32),
    )(f1nb, f2ns)


def _make_gather_pool_body(bias):
    def body(table_hbm, idx_hbm, out_hbm, idx_v, gidx_v, rows_v, out_v, sem):
        w = lax.axis_index("s") * NC + lax.axis_index("c")
        q0 = w * QW
        for t in range(K):
            pltpu.sync_copy(idx_hbm.at[pl.ds(t * QS + q0, QW)], idx_v.at[t])

        def chunk(co, carry):
            qo = co * CQ
            for t in range(K):
                gidx_v[pl.ds(t * CQ, CQ)] = idx_v[t, pl.ds(qo, CQ)] + bias
            pltpu.async_copy(table_hbm.at[gidx_v], rows_v, sem).wait()

            def one_query(qq, _c):
                for c in range(D // 16):
                    sl = pl.ds(c * 16, 16)
                    acc_m = rows_v[qq, sl]
                    acc_s = acc_m
                    for t in range(1, K):
                        v = rows_v[t * CQ + qq, sl]
                        acc_m = jnp.maximum(acc_m, v)
                        acc_s = acc_s + v
                    out_v[qq, sl] = acc_s * (1.0 / K)
                    out_v[qq, pl.ds(D + c * 16, 16)] = acc_m
                return _c

            lax.fori_loop(0, CQ, one_query, 0)
            pltpu.sync_copy(out_v, out_hbm.at[pl.ds(q0 + qo, CQ)])
            return carry

        lax.fori_loop(0, QW // CQ, chunk, 0)

    return body


def _gather_pool_slice(table_flat, idx_s, bias):
    f = functools.partial(
        pl.kernel,
        mesh=plsc.VectorSubcoreMesh(core_axis_name="c", subcore_axis_name="s"),
        out_type=jax.ShapeDtypeStruct((QS, 2 * D), jnp.float32),
        scratch_types=[
            pltpu.VMEM((K, QW), jnp.int32),
            pltpu.VMEM((K * CQ,), jnp.int32),
            pltpu.VMEM((K * CQ, D), jnp.float32),
            pltpu.VMEM((CQ, 2 * D), jnp.float32),
            pltpu.SemaphoreType.DMA,
        ],
    )(_make_gather_pool_body(bias))
    return f(table_flat, idx_s.reshape(K * QS))


def kernel(f1, f2, p, q, k):
    table = f1.reshape(B * NP, D)
    f1n = _normalize(table).reshape(B, NP, D)
    f2n = _normalize(f2.reshape(B * NQ, D)).reshape(B, NQ, D)
    idx_parts, agg_parts = [], []
    for b in range(B):
        for s in range(SLICES_PER_B):
            qo = s * QS
            idx_s = _topk_slice(f1n[b], f2n[b, qo:qo + QS])
            agg_s = _gather_pool_slice(table, idx_s, b * NP)
            idx_parts.append(idx_s)
            agg_parts.append(agg_s)
    idx_t = jnp.stack([jnp.concatenate(idx_parts[b * SLICES_PER_B:
                                                 (b + 1) * SLICES_PER_B], axis=1)
                       for b in range(B)])
    agg = jnp.concatenate(agg_parts, axis=0).reshape(B, NQ, 2 * D)
    out = jnp.concatenate([f2, agg], axis=-1)
    idx = jnp.transpose(idx_t, (0, 2, 1))
    return out, idx


# speedup vs baseline: 1.2475x; 1.2475x over previous
import functools

import jax
import jax.numpy as jnp
from jax import lax
from jax.experimental import pallas as pl
from jax.experimental.pallas import tpu as pltpu
from jax.experimental.pallas import tpu_sc as plsc

B, NP, NQ, D, K = 2, 4096, 4096, 256, 16
TQ = 256
SLICES_PER_B = 2
QS = NQ // SLICES_PER_B
NC, NS = 2, 16
NW = NC * NS
QW = QS // NW
CQ = 8
_PREC = lax.Precision.DEFAULT


def _normalize_body(x_ref, y_ref):
    x = x_ref[...]
    n = jnp.sqrt(jnp.sum(x * x, axis=1, keepdims=True))
    y_ref[...] = x / (n + 1e-8)


def _normalize(x):
    rows = x.shape[0]
    return pl.pallas_call(
        _normalize_body,
        grid=(rows // 1024,),
        in_specs=[pl.BlockSpec((1024, D), lambda i: (i, 0))],
        out_specs=pl.BlockSpec((1024, D), lambda i: (i, 0)),
        out_shape=jax.ShapeDtypeStruct((rows, D), jnp.float32),
    )(x)


def _topk_body(f1n_ref, f2n_ref, idx_ref):
    f1n = f1n_ref[...]
    f2n = f2n_ref[...]
    s = lax.dot_general(f1n, f2n, (((1,), (1,)), ((), ())),
                        preferred_element_type=jnp.float32,
                        precision=_PREC)
    rowiota = lax.broadcasted_iota(jnp.int32, (NP, TQ), 0)
    for t in range(K):
        sel = jnp.argmax(s, axis=0).astype(jnp.int32)
        idx_ref[t, :] = sel
        s = jnp.where(rowiota == sel[None, :], -jnp.inf, s)


def _topk_slice(f1nb, f2ns):
    return pl.pallas_call(
        _topk_body,
        grid=(QS // TQ,),
        in_specs=[
            pl.BlockSpec((NP, D), lambda i: (0, 0)),
            pl.BlockSpec((TQ, D), lambda i: (i, 0)),
        ],
        out_specs=pl.BlockSpec((K, TQ), lambda i: (0, i)),
        out_shape=jax.ShapeDtypeStruct((K, QS), jnp.int32),
    )(f1nb, f2ns)


_NCHUNK = QW // CQ


def _make_gather_pool_body(bias):
    def body(table_hbm, idx_hbm, out_hbm, idx_v, gidx0, gidx1, rows0, rows1,
             out_v, sem0, sem1):
        w = lax.axis_index("s") * NC + lax.axis_index("c")
        q0 = w * QW
        for t in range(K):
            pltpu.sync_copy(idx_hbm.at[pl.ds(t * QS + q0, QW)], idx_v.at[t])

        gidx = (gidx0, gidx1)
        rows = (rows0, rows1)
        sems = (sem0, sem1)

        def fill(co, buf):
            qo = co * CQ
            for t in range(K):
                gidx[buf][pl.ds(t * CQ, CQ)] = idx_v[t, pl.ds(qo, CQ)] + bias
            pltpu.async_copy(table_hbm.at[gidx[buf]], rows[buf], sems[buf])

        def drain(buf):
            pltpu.make_async_copy(table_hbm.at[gidx[buf]], rows[buf],
                                  sems[buf]).wait()

        def compute(co, buf):
            rv = rows[buf]

            def one_query(qq, _c):
                for c in range(D // 16):
                    sl = pl.ds(c * 16, 16)
                    acc_m = rv[qq, sl]
                    acc_s = acc_m
                    for t in range(1, K):
                        v = rv[t * CQ + qq, sl]
                        acc_m = jnp.maximum(acc_m, v)
                        acc_s = acc_s + v
                    out_v[qq, sl] = acc_s * (1.0 / K)
                    out_v[qq, pl.ds(D + c * 16, 16)] = acc_m
                return _c

            lax.fori_loop(0, CQ, one_query, 0)
            pltpu.sync_copy(out_v, out_hbm.at[pl.ds(q0 + co * CQ, CQ)])

        fill(0, 0)

        def pair(g, carry):
            c0 = g * 2
            fill(c0 + 1, 1)
            drain(0)
            compute(c0, 0)

            @pl.when(c0 + 2 < _NCHUNK)
            def _():
                fill(c0 + 2, 0)

            drain(1)
            compute(c0 + 1, 1)
            return carry

        lax.fori_loop(0, _NCHUNK // 2, pair, 0)

    return body


def _gather_pool_slice(table_flat, idx_s, bias):
    f = functools.partial(
        pl.kernel,
        mesh=plsc.VectorSubcoreMesh(core_axis_name="c", subcore_axis_name="s"),
        out_type=jax.ShapeDtypeStruct((QS, 2 * D), jnp.float32),
        scratch_types=[
            pltpu.VMEM((K, QW), jnp.int32),
            pltpu.VMEM((K * CQ,), jnp.int32),
            pltpu.VMEM((K * CQ,), jnp.int32),
            pltpu.VMEM((K * CQ, D), jnp.float32),
            pltpu.VMEM((K * CQ, D), jnp.float32),
            pltpu.VMEM((CQ, 2 * D), jnp.float32),
            pltpu.SemaphoreType.DMA,
            pltpu.SemaphoreType.DMA,
        ],
    )(_make_gather_pool_body(bias))
    return f(table_flat, idx_s.reshape(K * QS))


def kernel(f1, f2, p, q, k):
    table = f1.reshape(B * NP, D)
    f1n = _normalize(table).reshape(B, NP, D)
    f2n = _normalize(f2.reshape(B * NQ, D)).reshape(B, NQ, D)
    idx_parts, agg_parts = [], []
    for b in range(B):
        for s in range(SLICES_PER_B):
            qo = s * QS
            idx_s = _topk_slice(f1n[b], f2n[b, qo:qo + QS])
            agg_s = _gather_pool_slice(table, idx_s, b * NP)
            idx_parts.append(idx_s)
            agg_parts.append(agg_s)
    idx_t = jnp.stack([jnp.concatenate(idx_parts[b * SLICES_PER_B:
                                                 (b + 1) * SLICES_PER_B], axis=1)
                       for b in range(B)])
    agg = jnp.concatenate(agg_parts, axis=0).reshape(B, NQ, 2 * D)
    out = jnp.concatenate([f2, agg], axis=-1)
    idx = jnp.transpose(idx_t, (0, 2, 1))
    return out, idx
